# baseline (speedup 1.0000x reference)
import jax
import jax.numpy as jnp
from jax import lax
from jax.experimental import pallas as pl
from jax.experimental.pallas import tpu as pltpu
from jax.experimental.pallas import tpu_sc as plsc

M = 1000000
D = 16
B = 16384
L = 16
NC = 2
NS = 16
NW = NC * NS
RPW = 31256
LAST = M - (NW - 1) * RPW
NV = B // L
CPR = 512
NCP = 62
SPC = 17
NB = 4
SG = 8


def _body(mem_hbm, idx_hbm, val_hbm, out_hbm,
          idx_v, winner, dst_l, src_l, buf0, buf1, buf2, buf3,
          eidx, sidx, stage, ibuf, mbuf, lmbuf,
          in_sem0, in_sem1, in_sem2, in_sem3,
          out_sem0, out_sem1, out_sem2, out_sem3, idx_sem, g_sem, s_sem):
    cid = lax.axis_index("c")
    sid = lax.axis_index("s")
    wid = sid * NC + cid
    lo = pl.multiple_of(wid * RPW, 8)
    rows = jnp.where(wid == NW - 1, LAST, RPW)

    def cbase(c):
        return pl.multiple_of((lo + jnp.minimum(c * CPR, rows - CPR)) * D, 8)

    pltpu.async_copy(idx_hbm, idx_v, idx_sem).wait()

    iota = lax.iota(jnp.int32, L)

    def scan_body(j, n):
        iv = idx_v[pl.ds(j * L, L)]
        local = iv - lo
        m = (local >= 0) & (local < rows)
        jv = j * L + iota
        plsc.store_scatter(winner, [local], jv, mask=m)
        g = plsc.load_gather(winner, [local], mask=m)
        w1 = m & (g == jv)
        has_dup = jnp.sum(m.astype(jnp.int32)) != jnp.sum(w1.astype(jnp.int32))
        lmbuf[...] = w1.astype(jnp.int32)

        @pl.when(has_dup)
        def _():
            ibuf[pl.ds(0, L)] = iv
            ibuf[pl.ds(L, L)] = jnp.full((L,), -1, jnp.int32)
            mbuf[pl.ds(0, L)] = m.astype(jnp.int32)
            mbuf[pl.ds(L, L)] = jnp.zeros((L,), jnp.int32)
            loser = jnp.zeros((L,), jnp.bool_)
            for s in range(1, L):
                shiv = ibuf[pl.ds(s, L)]
                shm = mbuf[pl.ds(s, L)] != 0
                loser = loser | ((iv == shiv) & shm)
            lmm = m & jnp.logical_not(loser)
            plsc.store_scatter(winner, [local], jv, mask=lmm)
            lmbuf[...] = lmm.astype(jnp.int32)

        lm = lmbuf[...] != 0
        plsc.store_compressed(dst_l.at[pl.ds(n, L)], local, mask=lm)
        plsc.store_compressed(src_l.at[pl.ds(n, L)], jv, mask=lm)
        return n + jnp.sum(lm.astype(jnp.int32))

    bufs = ((buf0, in_sem0, out_sem0), (buf1, in_sem1, out_sem1),
            (buf2, in_sem2, out_sem2), (buf3, in_sem3, out_sem3))
    for p, (buf, isem, _) in enumerate(bufs):
        pltpu.async_copy(mem_hbm.at[pl.ds(cbase(p), CPR * D)], buf, isem)

    def copy_chunk(c, n):
        b = cbase(c)
        for p, (buf, isem, osem) in enumerate(bufs):
            @pl.when((c & (NB - 1)) == p)
            def _():
                pltpu.make_async_copy(mem_hbm.at[pl.ds(b, CPR * D)], buf,
                                      isem).wait()
                pltpu.async_copy(buf, out_hbm.at[pl.ds(b, CPR * D)], osem)

        n = lax.fori_loop(c * SPC, jnp.minimum((c + 1) * SPC, NV),
                          scan_body, n)

        for p, (buf, isem, osem) in enumerate(bufs):
            @pl.when((c & (NB - 1)) == p)
            def _():
                pltpu.make_async_copy(buf, out_hbm.at[pl.ds(b, CPR * D)],
                                      osem).wait()

                @pl.when(c + NB < NCP)
                def _():
                    pltpu.async_copy(
                        mem_hbm.at[pl.ds(cbase(c + NB), CPR * D)], buf, isem)

        return n

    n = lax.fori_loop(0, NCP, copy_chunk, jnp.int32(0))

    def filt_body(k, nf):
        dv = dst_l[pl.ds(k * L, L)]
        sv = src_l[pl.ds(k * L, L)]
        valid = (k * L + iota) < n
        w = plsc.load_gather(winner, [dv], mask=valid)
        keep = valid & (w == sv)
        plsc.store_compressed(dst_l.at[pl.ds(nf, L)], dv, mask=keep)
        plsc.store_compressed(src_l.at[pl.ds(nf, L)], sv, mask=keep)
        return nf + jnp.sum(keep.astype(jnp.int32))

    nf = lax.fori_loop(0, (n + L - 1) // L, filt_body, jnp.int32(0))

    @pl.when(nf > 0)
    def _():
        zeros16 = jnp.zeros((L,), jnp.int32)
        d0 = plsc.load_gather(dst_l, [zeros16])
        s0 = plsc.load_gather(src_l, [zeros16])

        @pl.when(nf % L != 0)
        def _():
            tv = (nf // L) * L
            posm = (tv + iota) < nf
            cd = dst_l[pl.ds(tv, L)]
            cs = src_l[pl.ds(tv, L)]
            dst_l[pl.ds(tv, L)] = jnp.where(posm, cd, d0)
            src_l[pl.ds(tv, L)] = jnp.where(posm, cs, s0)

        def sg_body(g, _):
            for e in range(SG):
                bc = jnp.full((L,), g * SG + e, jnp.int32)
                de = plsc.load_gather(dst_l, [bc])
                se = plsc.load_gather(src_l, [bc])
                eidx.at[0][pl.ds(e * D, D)] = (de + lo) * D + iota
                sidx.at[0][pl.ds(e * D, D)] = se * D + iota
            pltpu.async_copy(val_hbm.at[sidx.at[0]], stage, g_sem).wait()
            pltpu.async_copy(stage, out_hbm.at[eidx.at[0]], s_sem).wait()
            return 0

        lax.fori_loop(0, (nf + SG - 1) // SG, sg_body, 0)


_scatter_call = pl.kernel(
    _body,
    out_type=jax.ShapeDtypeStruct((M * D,), jnp.float32),
    mesh=plsc.VectorSubcoreMesh(core_axis_name="c", subcore_axis_name="s"),
    compiler_params=pltpu.CompilerParams(needs_layout_passes=False,
                                         use_tc_tiling_on_sc=False),
    scratch_types=[
        pltpu.VMEM((B,), jnp.int32),
        pltpu.VMEM((RPW + L,), jnp.int32),
        pltpu.VMEM((B + L,), jnp.int32),
        pltpu.VMEM((B + L,), jnp.int32),
        pltpu.VMEM((CPR * D,), jnp.float32),
        pltpu.VMEM((CPR * D,), jnp.float32),
        pltpu.VMEM((CPR * D,), jnp.float32),
        pltpu.VMEM((CPR * D,), jnp.float32),
        pltpu.VMEM((1, SG * D), jnp.int32),
        pltpu.VMEM((1, SG * D), jnp.int32),
        pltpu.VMEM((SG * D,), jnp.float32),
        pltpu.VMEM((2 * L,), jnp.int32),
        pltpu.VMEM((2 * L,), jnp.int32),
        pltpu.VMEM((L,), jnp.int32),
        pltpu.SemaphoreType.DMA,
        pltpu.SemaphoreType.DMA,
        pltpu.SemaphoreType.DMA,
        pltpu.SemaphoreType.DMA,
        pltpu.SemaphoreType.DMA,
        pltpu.SemaphoreType.DMA,
        pltpu.SemaphoreType.DMA,
        pltpu.SemaphoreType.DMA,
        pltpu.SemaphoreType.DMA,
        pltpu.SemaphoreType.DMA,
        pltpu.SemaphoreType.DMA,
    ],
)


def kernel(mem, idx, val):
    out1 = _scatter_call(mem.reshape(M * D),
                         idx.astype(jnp.int32),
                         val.reshape(B * D))
    return out1.reshape(M, D)


# speedup vs baseline: 1.0046x; 1.0046x over previous
import jax
import jax.numpy as jnp
from jax import lax
from jax.experimental import pallas as pl
from jax.experimental.pallas import tpu as pltpu
from jax.experimental.pallas import tpu_sc as plsc

M = 1000000
D = 16
B = 16384
L = 16
NC = 2
NS = 16
NW = NC * NS
RPW = 31256
LAST = M - (NW - 1) * RPW
NV = B // L
SG = 8


def _body(idx_hbm, val_hbm, out_hbm,
          idx_v, winner, dst_l, src_l, eidx, sidx, stage, ibuf, mbuf, lmbuf,
          idx_sem, g_sem, s_sem):
    cid = lax.axis_index("c")
    sid = lax.axis_index("s")
    wid = sid * NC + cid
    lo = pl.multiple_of(wid * RPW, 8)
    rows = jnp.where(wid == NW - 1, LAST, RPW)

    pltpu.async_copy(idx_hbm, idx_v, idx_sem).wait()

    iota = lax.iota(jnp.int32, L)

    def scan_body(j, n):
        iv = idx_v[pl.ds(j * L, L)]
        local = iv - lo
        m = (local >= 0) & (local < rows)
        jv = j * L + iota
        plsc.store_scatter(winner, [local], jv, mask=m)
        g = plsc.load_gather(winner, [local], mask=m)
        w1 = m & (g == jv)
        has_dup = jnp.sum(m.astype(jnp.int32)) != jnp.sum(w1.astype(jnp.int32))
        lmbuf[...] = w1.astype(jnp.int32)

        @pl.when(has_dup)
        def _():
            ibuf[pl.ds(0, L)] = iv
            ibuf[pl.ds(L, L)] = jnp.full((L,), -1, jnp.int32)
            mbuf[pl.ds(0, L)] = m.astype(jnp.int32)
            mbuf[pl.ds(L, L)] = jnp.zeros((L,), jnp.int32)
            loser = jnp.zeros((L,), jnp.bool_)
            for s in range(1, L):
                shiv = ibuf[pl.ds(s, L)]
                shm = mbuf[pl.ds(s, L)] != 0
                loser = loser | ((iv == shiv) & shm)
            lmm = m & jnp.logical_not(loser)
            plsc.store_scatter(winner, [local], jv, mask=lmm)
            lmbuf[...] = lmm.astype(jnp.int32)

        lm = lmbuf[...] != 0
        plsc.store_compressed(dst_l.at[pl.ds(n, L)], local, mask=lm)
        plsc.store_compressed(src_l.at[pl.ds(n, L)], jv, mask=lm)
        return n + jnp.sum(lm.astype(jnp.int32))

    n = lax.fori_loop(0, NV, scan_body, jnp.int32(0))

    def filt_body(k, nf):
        dv = dst_l[pl.ds(k * L, L)]
        sv = src_l[pl.ds(k * L, L)]
        valid = (k * L + iota) < n
        w = plsc.load_gather(winner, [dv], mask=valid)
        keep = valid & (w == sv)
        plsc.store_compressed(dst_l.at[pl.ds(nf, L)], dv, mask=keep)
        plsc.store_compressed(src_l.at[pl.ds(nf, L)], sv, mask=keep)
        return nf + jnp.sum(keep.astype(jnp.int32))

    nf = lax.fori_loop(0, (n + L - 1) // L, filt_body, jnp.int32(0))

    @pl.when(nf > 0)
    def _():
        zeros16 = jnp.zeros((L,), jnp.int32)
        d0 = plsc.load_gather(dst_l, [zeros16])
        s0 = plsc.load_gather(src_l, [zeros16])

        @pl.when(nf % L != 0)
        def _():
            tv = (nf // L) * L
            posm = (tv + iota) < nf
            cd = dst_l[pl.ds(tv, L)]
            cs = src_l[pl.ds(tv, L)]
            dst_l[pl.ds(tv, L)] = jnp.where(posm, cd, d0)
            src_l[pl.ds(tv, L)] = jnp.where(posm, cs, s0)

        def sg_body(g, _):
            for e in range(SG):
                bc = jnp.full((L,), g * SG + e, jnp.int32)
                de = plsc.load_gather(dst_l, [bc])
                se = plsc.load_gather(src_l, [bc])
                eidx.at[0][pl.ds(e * D, D)] = (de + lo) * D + iota
                sidx.at[0][pl.ds(e * D, D)] = se * D + iota
            pltpu.async_copy(val_hbm.at[sidx.at[0]], stage, g_sem).wait()
            pltpu.async_copy(stage, out_hbm.at[eidx.at[0]], s_sem).wait()
            return 0

        lax.fori_loop(0, (nf + SG - 1) // SG, sg_body, 0)


_scatter_call = pl.kernel(
    _body,
    out_type=(),
    mesh=plsc.VectorSubcoreMesh(core_axis_name="c", subcore_axis_name="s"),
    compiler_params=pltpu.CompilerParams(needs_layout_passes=False,
                                         use_tc_tiling_on_sc=False),
    scratch_types=[
        pltpu.VMEM((B,), jnp.int32),
        pltpu.VMEM((RPW + L,), jnp.int32),
        pltpu.VMEM((B + L,), jnp.int32),
        pltpu.VMEM((B + L,), jnp.int32),
        pltpu.VMEM((1, SG * D), jnp.int32),
        pltpu.VMEM((1, SG * D), jnp.int32),
        pltpu.VMEM((SG * D,), jnp.float32),
        pltpu.VMEM((2 * L,), jnp.int32),
        pltpu.VMEM((2 * L,), jnp.int32),
        pltpu.VMEM((L,), jnp.int32),
        pltpu.SemaphoreType.DMA,
        pltpu.SemaphoreType.DMA,
        pltpu.SemaphoreType.DMA,
    ],
)


def kernel(mem, idx, val):
    out_ref = jax.new_ref(mem.reshape(M * D))
    _scatter_call(idx.astype(jnp.int32), val.reshape(B * D), out_ref)
    return jax.freeze(out_ref).reshape(M, D)


# speedup vs baseline: 1.0079x; 1.0032x over previous
import jax
import jax.numpy as jnp
from jax import lax
from jax.experimental import pallas as pl
from jax.experimental.pallas import tpu as pltpu
from jax.experimental.pallas import tpu_sc as plsc

M = 1000000
D = 16
B = 16384
L = 16
NC = 2
NS = 16
NW = NC * NS
RPW = 31256
LAST = M - (NW - 1) * RPW
NV = B // L
SG = 8


def _body(idx_hbm, val_hbm, out_hbm,
          idx_v, winner, dst_l, src_l, eidx, sidx, stage, ibuf, mbuf, lmbuf,
          cbuf, idx_sem, g_sem, s_sem):
    cid = lax.axis_index("c")
    sid = lax.axis_index("s")
    wid = sid * NC + cid
    lo = pl.multiple_of(wid * RPW, 8)
    rows = jnp.where(wid == NW - 1, LAST, RPW)

    pltpu.async_copy(idx_hbm, idx_v, idx_sem).wait()

    iota = lax.iota(jnp.int32, L)

    def scan_body(j, n):
        iv = idx_v[pl.ds(j * L, L)]
        local = iv - lo
        m = (local >= 0) & (local < rows)
        jv = j * L + iota
        nm = jnp.sum(m.astype(jnp.int32))
        cbuf[0] = jnp.int32(0)

        @pl.when(nm > 0)
        def _():
            _scan_nonempty(n, iv, local, m, jv, nm)

        return n + cbuf[0]

    def _scan_nonempty(n, iv, local, m, jv, nm):
        plsc.store_scatter(winner, [local], jv, mask=m)
        g = plsc.load_gather(winner, [local], mask=m)
        w1 = m & (g == jv)
        has_dup = nm != jnp.sum(w1.astype(jnp.int32))
        lmbuf[...] = w1.astype(jnp.int32)

        @pl.when(has_dup)
        def _():
            ibuf[pl.ds(0, L)] = iv
            ibuf[pl.ds(L, L)] = jnp.full((L,), -1, jnp.int32)
            mbuf[pl.ds(0, L)] = m.astype(jnp.int32)
            mbuf[pl.ds(L, L)] = jnp.zeros((L,), jnp.int32)
            loser = jnp.zeros((L,), jnp.bool_)
            for s in range(1, L):
                shiv = ibuf[pl.ds(s, L)]
                shm = mbuf[pl.ds(s, L)] != 0
                loser = loser | ((iv == shiv) & shm)
            lmm = m & jnp.logical_not(loser)
            plsc.store_scatter(winner, [local], jv, mask=lmm)
            lmbuf[...] = lmm.astype(jnp.int32)

        lm = lmbuf[...] != 0
        plsc.store_compressed(dst_l.at[pl.ds(n, L)], local, mask=lm)
        plsc.store_compressed(src_l.at[pl.ds(n, L)], jv, mask=lm)
        cbuf[0] = jnp.sum(lm.astype(jnp.int32))

    n = lax.fori_loop(0, NV, scan_body, jnp.int32(0))

    def filt_body(k, nf):
        dv = dst_l[pl.ds(k * L, L)]
        sv = src_l[pl.ds(k * L, L)]
        valid = (k * L + iota) < n
        w = plsc.load_gather(winner, [dv], mask=valid)
        keep = valid & (w == sv)
        plsc.store_compressed(dst_l.at[pl.ds(nf, L)], dv, mask=keep)
        plsc.store_compressed(src_l.at[pl.ds(nf, L)], sv, mask=keep)
        return nf + jnp.sum(keep.astype(jnp.int32))

    nf = lax.fori_loop(0, (n + L - 1) // L, filt_body, jnp.int32(0))

    @pl.when(nf > 0)
    def _():
        zeros16 = jnp.zeros((L,), jnp.int32)
        d0 = plsc.load_gather(dst_l, [zeros16])
        s0 = plsc.load_gather(src_l, [zeros16])

        @pl.when(nf % L != 0)
        def _():
            tv = (nf // L) * L
            posm = (tv + iota) < nf
            cd = dst_l[pl.ds(tv, L)]
            cs = src_l[pl.ds(tv, L)]
            dst_l[pl.ds(tv, L)] = jnp.where(posm, cd, d0)
            src_l[pl.ds(tv, L)] = jnp.where(posm, cs, s0)

        def sg_body(g, _):
            for e in range(SG):
                bc = jnp.full((L,), g * SG + e, jnp.int32)
                de = plsc.load_gather(dst_l, [bc])
                se = plsc.load_gather(src_l, [bc])
                eidx.at[0][pl.ds(e * D, D)] = (de + lo) * D + iota
                sidx.at[0][pl.ds(e * D, D)] = se * D + iota
            pltpu.async_copy(val_hbm.at[sidx.at[0]], stage, g_sem).wait()
            pltpu.async_copy(stage, out_hbm.at[eidx.at[0]], s_sem).wait()
            return 0

        lax.fori_loop(0, (nf + SG - 1) // SG, sg_body, 0)


_scatter_call = pl.kernel(
    _body,
    out_type=(),
    mesh=plsc.VectorSubcoreMesh(core_axis_name="c", subcore_axis_name="s"),
    compiler_params=pltpu.CompilerParams(needs_layout_passes=False,
                                         use_tc_tiling_on_sc=False),
    scratch_types=[
        pltpu.VMEM((B,), jnp.int32),
        pltpu.VMEM((RPW + L,), jnp.int32),
        pltpu.VMEM((B + L,), jnp.int32),
        pltpu.VMEM((B + L,), jnp.int32),
        pltpu.VMEM((1, SG * D), jnp.int32),
        pltpu.VMEM((1, SG * D), jnp.int32),
        pltpu.VMEM((SG * D,), jnp.float32),
        pltpu.VMEM((2 * L,), jnp.int32),
        pltpu.VMEM((2 * L,), jnp.int32),
        pltpu.VMEM((L,), jnp.int32),
        pltpu.SMEM((1,), jnp.int32),
        pltpu.SemaphoreType.DMA,
        pltpu.SemaphoreType.DMA,
        pltpu.SemaphoreType.DMA,
    ],
)


def kernel(mem, idx, val):
    out_ref = jax.new_ref(mem.reshape(M * D))
    _scatter_call(idx.astype(jnp.int32), val.reshape(B * D), out_ref)
    return jax.freeze(out_ref).reshape(M, D)
